# baseline (speedup 1.0000x reference)
import functools

import jax
import jax.numpy as jnp
from jax import lax
from jax.experimental import pallas as pl
from jax.experimental.pallas import tpu as pltpu
from jax.experimental.pallas import tpu_sc as plsc

_VOCAB = 100000
_EMB = 64
_B = 1024
_L = 50

_NC = 2
_NS = 16
_NW = _NC * _NS
_SAMP_PER_W = _B // _NW
_CHUNK = 100
_CHUNKS_PER_W = _SAMP_PER_W * _L // _CHUNK
_LANES = 16


def _pool_body(idx_hbm, table_hbm, out_hbm, idx_v, rows_v, out_v, sem):
    wid = lax.axis_index("s") * _NC + lax.axis_index("c")
    pltpu.sync_copy(idx_hbm.at[pl.ds(wid * _CHUNKS_PER_W, _CHUNKS_PER_W)], idx_v)
    copies = []
    for j in range(_CHUNKS_PER_W):
        copies.append(
            pltpu.async_copy(
                table_hbm.at[idx_v.at[j]],
                rows_v.at[pl.ds(j * _CHUNK, _CHUNK)],
                sem,
            )
        )
    for cp in copies:
        cp.wait()

    scale = jnp.float32(1.0 / _L)

    def sample_body(s, carry):
        base = s * _L
        acc = [jnp.zeros((_LANES,), jnp.float32) for _ in range(_EMB // _LANES)]
        for l in range(_L):
            r = base + l
            for k in range(_EMB // _LANES):
                acc[k] = acc[k] + rows_v[r, pl.ds(k * _LANES, _LANES)]
        for k in range(_EMB // _LANES):
            out_v[s, pl.ds(k * _LANES, _LANES)] = acc[k] * scale
        return carry

    lax.fori_loop(0, _SAMP_PER_W, sample_body, jnp.int32(0))
    pltpu.sync_copy(out_v, out_hbm.at[pl.ds(wid * _SAMP_PER_W, _SAMP_PER_W)])


_pool = functools.partial(
    pl.kernel,
    out_type=jax.ShapeDtypeStruct((_B, _EMB), jnp.float32),
    mesh=plsc.VectorSubcoreMesh(core_axis_name="c", subcore_axis_name="s"),
    scratch_types=[
        pltpu.VMEM((_CHUNKS_PER_W, _CHUNK), jnp.int32),
        pltpu.VMEM((_SAMP_PER_W * _L, _EMB), jnp.float32),
        pltpu.VMEM((_SAMP_PER_W, _EMB), jnp.float32),
        pltpu.SemaphoreType.DMA,
    ],
    compiler_params=pltpu.CompilerParams(use_tc_tiling_on_sc=False),
)(_pool_body)


_NBLK = 512
_GRID_N = (_VOCAB + _NBLK - 1) // _NBLK


def _proj_body(x_ref, w_ref, b_ref, o_ref):
    o_ref[...] = (
        jnp.dot(x_ref[...], w_ref[...], preferred_element_type=jnp.float32)
        + b_ref[...]
    )


def _project(x, W, b2):
    return pl.pallas_call(
        _proj_body,
        grid=(_GRID_N,),
        in_specs=[
            pl.BlockSpec((_B, _EMB), lambda i: (0, 0)),
            pl.BlockSpec((_EMB, _NBLK), lambda i: (0, i)),
            pl.BlockSpec((1, _NBLK), lambda i: (0, i)),
        ],
        out_specs=pl.BlockSpec((_B, _NBLK), lambda i: (0, i)),
        out_shape=jax.ShapeDtypeStruct((_B, _VOCAB), jnp.float32),
        compiler_params=pltpu.CompilerParams(
            dimension_semantics=("arbitrary",),
        ),
    )(x, W, b2)


def kernel(inputs, emb_table, W, b):
    idx2d = inputs.reshape(_B * _L // _CHUNK, _CHUNK)
    x = _pool(idx2d, emb_table)
    return _project(x, W, b.reshape(1, _VOCAB))


# speedup vs baseline: 1.1251x; 1.1251x over previous
import functools

import jax
import jax.numpy as jnp
from jax import lax
from jax.experimental import pallas as pl
from jax.experimental.pallas import tpu as pltpu
from jax.experimental.pallas import tpu_sc as plsc

_VOCAB = 100000
_EMB = 64
_B = 1024
_L = 50

_NC = 2
_NS = 16
_NW = _NC * _NS
_SAMP_PER_W = _B // _NW
_CHUNK = 100
_CHUNKS_PER_W = _SAMP_PER_W * _L // _CHUNK
_LANES = 16


def _pool_body(idx_hbm, table_hbm, out_hbm, idx_v, rows_v, out_v, sem):
    wid = lax.axis_index("s") * _NC + lax.axis_index("c")
    pltpu.sync_copy(idx_hbm.at[pl.ds(wid * _CHUNKS_PER_W, _CHUNKS_PER_W)], idx_v)
    copies = []
    for j in range(_CHUNKS_PER_W):
        copies.append(
            pltpu.async_copy(
                table_hbm.at[idx_v.at[j]],
                rows_v.at[pl.ds(j * _CHUNK, _CHUNK)],
                sem,
            )
        )
    for cp in copies:
        cp.wait()

    scale = jnp.float32(1.0 / _L)

    def sample_body(s, carry):
        base = s * _L
        acc = [jnp.zeros((_LANES,), jnp.float32) for _ in range(_EMB // _LANES)]
        for l in range(_L):
            r = base + l
            for k in range(_EMB // _LANES):
                acc[k] = acc[k] + rows_v[r, pl.ds(k * _LANES, _LANES)]
        for k in range(_EMB // _LANES):
            out_v[s, pl.ds(k * _LANES, _LANES)] = acc[k] * scale
        return carry

    lax.fori_loop(0, _SAMP_PER_W, sample_body, jnp.int32(0))
    pltpu.sync_copy(out_v, out_hbm.at[pl.ds(wid * _SAMP_PER_W, _SAMP_PER_W)])


_pool = functools.partial(
    pl.kernel,
    out_type=jax.ShapeDtypeStruct((_B, _EMB), jnp.float32),
    mesh=plsc.VectorSubcoreMesh(core_axis_name="c", subcore_axis_name="s"),
    scratch_types=[
        pltpu.VMEM((_CHUNKS_PER_W, _CHUNK), jnp.int32),
        pltpu.VMEM((_SAMP_PER_W * _L, _EMB), jnp.float32),
        pltpu.VMEM((_SAMP_PER_W, _EMB), jnp.float32),
        pltpu.SemaphoreType.DMA,
    ],
    compiler_params=pltpu.CompilerParams(use_tc_tiling_on_sc=False),
)(_pool_body)


_NBLK = 2048
_GRID_N = (_VOCAB + _NBLK - 1) // _NBLK
_TAIL = _VOCAB - (_GRID_N - 1) * _NBLK
_RSPLIT = 4
_RB = _B // _RSPLIT


def _proj_body(x_ref, w_ref, b_ref, out_ref, acc_ref, tail_ref, sem_ref, tsem_ref):
    i = pl.program_id(0)
    slot = lax.rem(i, 2)

    def _out_copy(src_slot, blk):
        copies = []
        for r in range(_RSPLIT):
            copies.append(
                pltpu.make_async_copy(
                    acc_ref.at[src_slot, pl.ds(r * _RB, _RB)],
                    out_ref.at[pl.ds(r * _RB, _RB), pl.ds(blk * _NBLK, _NBLK)],
                    sem_ref.at[src_slot, r],
                )
            )
        return copies

    def _tail_copy():
        return pltpu.make_async_copy(
            tail_ref,
            out_ref.at[:, pl.ds((_GRID_N - 1) * _NBLK, _TAIL)],
            tsem_ref,
        )

    @pl.when(i >= 2)
    def _():
        for cp in _out_copy(slot, i - 2):
            cp.wait()

    res = (
        jnp.dot(x_ref[...], w_ref[...], preferred_element_type=jnp.float32)
        + b_ref[...]
    )

    @pl.when(i < _GRID_N - 1)
    def _():
        acc_ref[slot] = res
        for cp in _out_copy(slot, i):
            cp.start()

    @pl.when(i == _GRID_N - 1)
    def _():
        tail_ref[...] = res[:, :_TAIL]
        _tail_copy().start()
        for cp in _out_copy(1 - slot, i - 1):
            cp.wait()
        _tail_copy().wait()


def _project(x, W, b2):
    return pl.pallas_call(
        _proj_body,
        grid=(_GRID_N,),
        in_specs=[
            pl.BlockSpec((_B, _EMB), lambda i: (0, 0)),
            pl.BlockSpec((_EMB, _NBLK), lambda i: (0, i)),
            pl.BlockSpec((1, _NBLK), lambda i: (0, i)),
        ],
        out_specs=pl.BlockSpec(memory_space=pl.ANY),
        out_shape=jax.ShapeDtypeStruct((_B, _VOCAB), jnp.float32),
        scratch_shapes=[
            pltpu.VMEM((2, _B, _NBLK), jnp.float32),
            pltpu.VMEM((_B, _TAIL), jnp.float32),
            pltpu.SemaphoreType.DMA((2, _RSPLIT)),
            pltpu.SemaphoreType.DMA,
        ],
        compiler_params=pltpu.CompilerParams(
            dimension_semantics=("arbitrary",),
        ),
    )(x, W, b2)


def kernel(inputs, emb_table, W, b):
    idx2d = inputs.reshape(_B * _L // _CHUNK, _CHUNK)
    x = _pool(idx2d, emb_table)
    return _project(x, W, b.reshape(1, _VOCAB))


# speedup vs baseline: 2.8816x; 2.5613x over previous
import functools

import jax
import jax.numpy as jnp
from jax import lax
from jax.experimental import pallas as pl
from jax.experimental.pallas import tpu as pltpu
from jax.experimental.pallas import tpu_sc as plsc

_VOCAB = 100000
_EMB = 64
_B = 1024
_L = 50

_NC = 2
_NS = 16
_NW = _NC * _NS
_SAMP_PER_W = _B // _NW
_CHUNK = 100
_CHUNKS_PER_W = _SAMP_PER_W * _L // _CHUNK
_LANES = 16


def _pool_body(idx_hbm, table_hbm, out_hbm, idx_v, rows_v, out_v, sem):
    wid = lax.axis_index("s") * _NC + lax.axis_index("c")
    pltpu.sync_copy(idx_hbm.at[pl.ds(wid * _CHUNKS_PER_W, _CHUNKS_PER_W)], idx_v)
    copies = []
    for j in range(_CHUNKS_PER_W):
        copies.append(
            pltpu.async_copy(
                table_hbm.at[idx_v.at[j]],
                rows_v.at[pl.ds(j * _CHUNK, _CHUNK)],
                sem,
            )
        )
    for cp in copies:
        cp.wait()

    scale = jnp.float32(1.0 / _L)

    def sample_body(s, carry):
        base = s * _L
        acc = [jnp.zeros((_LANES,), jnp.float32) for _ in range(_EMB // _LANES)]
        for l in range(_L):
            r = base + l
            for k in range(_EMB // _LANES):
                acc[k] = acc[k] + rows_v[r, pl.ds(k * _LANES, _LANES)]
        for k in range(_EMB // _LANES):
            out_v[s, pl.ds(k * _LANES, _LANES)] = acc[k] * scale
        return carry

    lax.fori_loop(0, _SAMP_PER_W, sample_body, jnp.int32(0))
    pltpu.sync_copy(out_v, out_hbm.at[pl.ds(wid * _SAMP_PER_W, _SAMP_PER_W)])


_pool = functools.partial(
    pl.kernel,
    out_type=jax.ShapeDtypeStruct((_B, _EMB), jnp.float32),
    mesh=plsc.VectorSubcoreMesh(core_axis_name="c", subcore_axis_name="s"),
    scratch_types=[
        pltpu.VMEM((_CHUNKS_PER_W, _CHUNK), jnp.int32),
        pltpu.VMEM((_SAMP_PER_W * _L, _EMB), jnp.float32),
        pltpu.VMEM((_SAMP_PER_W, _EMB), jnp.float32),
        pltpu.SemaphoreType.DMA,
    ],
    compiler_params=pltpu.CompilerParams(use_tc_tiling_on_sc=False),
)(_pool_body)


_NBLK = 2048
_GRID_N = (_VOCAB + _NBLK - 1) // _NBLK
_TAIL = _VOCAB - (_GRID_N - 1) * _NBLK
_RSPLIT = 4
_RB = _NBLK // _RSPLIT
_RT = _TAIL // _RSPLIT


def _proj_body(x_ref, w_ref, b_ref, out_ref, acc_ref, sem_ref):
    i = pl.program_id(0)
    slot = lax.rem(i, 2)

    def _out_copy(src_slot, blk, rows_per_copy):
        copies = []
        for r in range(_RSPLIT):
            copies.append(
                pltpu.make_async_copy(
                    acc_ref.at[src_slot, pl.ds(r * rows_per_copy, rows_per_copy)],
                    out_ref.at[pl.ds(blk * _NBLK + r * rows_per_copy, rows_per_copy)],
                    sem_ref.at[src_slot, r],
                )
            )
        return copies

    @pl.when(i >= 2)
    def _():
        for cp in _out_copy(slot, i - 2, _RB):
            cp.wait()

    acc_ref[slot] = lax.dot_general(
        w_ref[...], x_ref[...],
        dimension_numbers=(((0,), (1,)), ((), ())),
        preferred_element_type=jnp.float32,
    ) + b_ref[...][:, None]

    @pl.when(i < _GRID_N - 1)
    def _():
        for cp in _out_copy(slot, i, _RB):
            cp.start()

    @pl.when(i == _GRID_N - 1)
    def _():
        for cp in _out_copy(slot, i, _RT):
            cp.start()
        for cp in _out_copy(1 - slot, i - 1, _RB):
            cp.wait()
        for cp in _out_copy(slot, i, _RT):
            cp.wait()


def _project(x, W, b):
    logits_t = pl.pallas_call(
        _proj_body,
        grid=(_GRID_N,),
        in_specs=[
            pl.BlockSpec((_B, _EMB), lambda i: (0, 0)),
            pl.BlockSpec((_EMB, _NBLK), lambda i: (0, i)),
            pl.BlockSpec((_NBLK,), lambda i: (i,)),
        ],
        out_specs=pl.BlockSpec(memory_space=pl.ANY),
        out_shape=jax.ShapeDtypeStruct((_VOCAB, _B), jnp.float32),
        scratch_shapes=[
            pltpu.VMEM((2, _NBLK, _B), jnp.float32),
            pltpu.SemaphoreType.DMA((2, _RSPLIT)),
        ],
        compiler_params=pltpu.CompilerParams(
            dimension_semantics=("arbitrary",),
        ),
    )(x, W, b)
    return logits_t.T


def kernel(inputs, emb_table, W, b):
    idx2d = inputs.reshape(_B * _L // _CHUNK, _CHUNK)
    x = _pool(idx2d, emb_table)
    return _project(x, W, b)
